# Initial kernel scaffold; baseline (speedup 1.0000x reference)
import jax
import jax.numpy as jnp
from jax.experimental import pallas as pl


def kernel(hidden_states, gate_w, w1, w3, w2, sw1, sw3, sw2, shared_gate_w):
    raise NotImplementedError("write your pallas kernel here")


# speedup vs baseline: 3.2580x; 3.2580x over previous
import jax
import jax.numpy as jnp
from jax.experimental import pallas as pl
from jax.experimental.pallas import tpu as pltpu

T = 2048
D = 1024
E = 64
K = 8
F_MOE = 1408
F_SH = 2816
CAP = 512

_BF = jnp.bfloat16
_F32 = jnp.float32

_TCH = 512
_FCH = (768, 640)


def _routing_kernel(x_ref, gw_ref, gates_ref, rank_ref):
    x = x_ref[...]
    logits = jnp.dot(x, gw_ref[...], precision=jax.lax.Precision.HIGHEST)
    m = jnp.max(logits, axis=1, keepdims=True)
    p = jnp.exp(logits - m)
    p = p / jnp.sum(p, axis=1, keepdims=True)

    work = p
    kth = None
    for _ in range(K):
        kth = jnp.max(work, axis=1, keepdims=True)
        work = jnp.where(work == kth, -1.0, work)
    gates = jnp.where(p >= kth, p, 0.0)

    maskb = (gates > 0.0).astype(_BF)
    blk = 256
    row_i = jax.lax.broadcasted_iota(jnp.int32, (blk, blk), 0)
    col_i = jax.lax.broadcasted_iota(jnp.int32, (blk, blk), 1)
    ltri = (col_i <= row_i).astype(_BF)
    carry = jnp.zeros((1, E), dtype=_F32)
    chunks = []
    for c in range(T // blk):
        mc = maskb[c * blk:(c + 1) * blk, :]
        rc = jnp.dot(ltri, mc, preferred_element_type=_F32) + carry
        carry = carry + jnp.sum(mc.astype(_F32), axis=0, keepdims=True)
        chunks.append(rc)
    rank = jnp.concatenate(chunks, axis=0)
    rank = jnp.where(gates > 0.0, rank, 0.0)

    gates_ref[...] = gates
    rank_ref[...] = rank


def _expert_kernel(xb_ref, gates_ref, rank_ref, w1_ref, w3_ref, w2_ref,
                   out_ref):
    e = pl.program_id(0)
    lane = jax.lax.broadcasted_iota(jnp.int32, (1, E), 1)
    sel = lane == e
    g_col = jnp.sum(jnp.where(sel, gates_ref[...], 0.0), axis=1, keepdims=True)
    r_col = jnp.sum(jnp.where(sel, rank_ref[...], 0.0), axis=1, keepdims=True)
    r_i = r_col.astype(jnp.int32)

    cap_i = jax.lax.broadcasted_iota(jnp.int32, (_TCH, CAP), 1) + 1

    xe = jnp.zeros((CAP, D), dtype=_F32)
    for tb in range(T // _TCH):
        sl = slice(tb * _TCH, (tb + 1) * _TCH)
        ptc = (r_i[sl] == cap_i).astype(_BF)
        xe = xe + jax.lax.dot_general(
            ptc, xb_ref[sl], (((0,), (0,)), ((), ())),
            preferred_element_type=_F32)
    xe = xe.astype(_BF)

    y = jnp.zeros((CAP, D), dtype=_F32)
    lo = 0
    for sz in _FCH:
        w1h = w1_ref[0, :, lo:lo + sz].astype(_BF)
        a = jnp.dot(xe, w1h, preferred_element_type=_F32)
        w3h = w3_ref[0, :, lo:lo + sz].astype(_BF)
        b = jnp.dot(xe, w3h, preferred_element_type=_F32)
        h = (a * jax.nn.sigmoid(a) * b).astype(_BF)
        w2h = w2_ref[0, lo:lo + sz, :].astype(_BF)
        y = y + jnp.dot(h, w2h, preferred_element_type=_F32)
        lo += sz
    yb = y.astype(_BF)

    for tb in range(T // _TCH):
        sl = slice(tb * _TCH, (tb + 1) * _TCH)
        ptc = (r_i[sl] == cap_i).astype(_BF)
        contrib = jnp.dot(ptc, yb, preferred_element_type=_F32) * g_col[sl]

        @pl.when(e == 0)
        def _(sl=sl, contrib=contrib):
            out_ref[sl] = contrib

        @pl.when(e != 0)
        def _(sl=sl, contrib=contrib):
            out_ref[sl] = out_ref[sl] + contrib


def _shared_kernel(xb_ref, routed_ref, sw1_ref, sw3_ref, sw2_ref, sgw_ref,
                   out_ref, acc_ref):
    j = pl.program_id(1)
    nj = pl.num_programs(1)
    xb = xb_ref[...]
    a = jnp.dot(xb, sw1_ref[...].astype(_BF), preferred_element_type=_F32)
    b = jnp.dot(xb, sw3_ref[...].astype(_BF), preferred_element_type=_F32)
    h = (a * jax.nn.sigmoid(a) * b).astype(_BF)
    part = jnp.dot(h, sw2_ref[...].astype(_BF), preferred_element_type=_F32)

    @pl.when(j == 0)
    def _():
        acc_ref[...] = part

    @pl.when(j != 0)
    def _():
        acc_ref[...] = acc_ref[...] + part

    @pl.when(j == nj - 1)
    def _():
        sg = jnp.dot(xb, sgw_ref[...].astype(_BF), preferred_element_type=_F32)
        out_ref[...] = routed_ref[...] + acc_ref[...] * jax.nn.sigmoid(sg)


def kernel(hidden_states, gate_w, w1, w3, w2, sw1, sw3, sw2, shared_gate_w):
    orig_shape = hidden_states.shape
    x = hidden_states.reshape(-1, D)

    gates, rank = pl.pallas_call(
        _routing_kernel,
        out_shape=(
            jax.ShapeDtypeStruct((T, E), _F32),
            jax.ShapeDtypeStruct((T, E), _F32),
        ),
    )(x, gate_w)

    xb = x.astype(_BF)

    routed = pl.pallas_call(
        _expert_kernel,
        grid=(E,),
        in_specs=[
            pl.BlockSpec((T, D), lambda e: (0, 0)),
            pl.BlockSpec((T, E), lambda e: (0, 0)),
            pl.BlockSpec((T, E), lambda e: (0, 0)),
            pl.BlockSpec((1, D, F_MOE), lambda e: (e, 0, 0)),
            pl.BlockSpec((1, D, F_MOE), lambda e: (e, 0, 0)),
            pl.BlockSpec((1, F_MOE, D), lambda e: (e, 0, 0)),
        ],
        out_specs=pl.BlockSpec((T, D), lambda e: (0, 0)),
        out_shape=jax.ShapeDtypeStruct((T, D), _F32),
    )(xb, gates, rank, w1, w3, w2)

    tbs = 512
    out = pl.pallas_call(
        _shared_kernel,
        grid=(T // tbs, 2),
        in_specs=[
            pl.BlockSpec((tbs, D), lambda t, j: (t, 0)),
            pl.BlockSpec((tbs, D), lambda t, j: (t, 0)),
            pl.BlockSpec((D, F_SH // 2), lambda t, j: (0, j)),
            pl.BlockSpec((D, F_SH // 2), lambda t, j: (0, j)),
            pl.BlockSpec((F_SH // 2, D), lambda t, j: (j, 0)),
            pl.BlockSpec((D, 1), lambda t, j: (0, 0)),
        ],
        out_specs=pl.BlockSpec((tbs, D), lambda t, j: (t, 0)),
        out_shape=jax.ShapeDtypeStruct((T, D), _F32),
        scratch_shapes=[pltpu.VMEM((tbs, D), _F32)],
    )(xb, routed, sw1, sw3, sw2, shared_gate_w)

    return out.reshape(orig_shape)
